# Initial kernel scaffold; baseline (speedup 1.0000x reference)
import jax
import jax.numpy as jnp
from jax.experimental import pallas as pl


def kernel(users_features, items_features, user_id_table, user_feat_tables, user_bias, item_id_table, item_feat_tables, item_bias):
    raise NotImplementedError("write your pallas kernel here")


# speedup vs baseline: 1.0748x; 1.0748x over previous
import functools

import jax
import jax.numpy as jnp
from jax import lax
from jax.experimental import pallas as pl
from jax.experimental.pallas import tpu as pltpu
from jax.experimental.pallas import tpu_sc as plsc

B = 16384
V = 100000
D_ID = 64
D_F = 8
NF = 12
NCOL = 1 + NF

NC = 2
NS = 16
NW = NC * NS
L = 16

CPW = B // NW
NB = 4
CB = CPW // NB
FPB = CB * NF
ICH = 128


def _nes_body(uf_hbm, if_hbm, uid_tab, ufeat_tab, ubias, iid_tab, ifeat_tab,
              ibias, out_hbm, uf_v, if_v, uid_idx, iid_idx, ubh_idx, ibh_idx,
              ufidx, ifidx, uid_rows, iid_rows, uf_rows, if_rows, ub_rows,
              ib_rows, out_v, sem):
    wid = lax.axis_index("s") * NC + lax.axis_index("c")
    base = wid * CPW

    iota = lax.iota(jnp.int32, L)
    zeros = iota * 0
    i_d8 = iota // 8
    i_m8 = iota % 8
    m15 = iota == 15

    pltpu.sync_copy(uf_hbm.at[pl.ds(base, CPW)], uf_v)
    pltpu.sync_copy(if_hbm.at[pl.ds(base, CPW)], if_v)

    for g in range(NB):
        e0 = g * CB

        def id_idx_body(t, _):
            r = e0 + t * L + iota
            u = plsc.load_gather(uf_v, [r, zeros])
            iv = plsc.load_gather(if_v, [r, zeros])
            uid_idx[pl.ds(t * L, L)] = u
            iid_idx[pl.ds(t * L, L)] = iv
            ubh_idx[pl.ds(t * L, L)] = u >> 3
            ibh_idx[pl.ds(t * L, L)] = iv >> 3
            return 0

        lax.fori_loop(0, CB // L, id_idx_body, 0, unroll=2)

        def f_idx_body(t, _):
            k = t * L + iota
            e = k // NF
            i = k - e * NF
            r = e0 + e
            c = 1 + i
            off = i * V
            ufidx[pl.ds(t * L, L)] = plsc.load_gather(uf_v, [r, c]) + off
            ifidx[pl.ds(t * L, L)] = plsc.load_gather(if_v, [r, c]) + off
            return 0

        lax.fori_loop(0, FPB // L, f_idx_body, 0, unroll=2)

        copies = [
            pltpu.async_copy(uid_tab.at[uid_idx], uid_rows, sem),
            pltpu.async_copy(iid_tab.at[iid_idx], iid_rows, sem),
            pltpu.async_copy(ubias.at[ubh_idx], ub_rows, sem),
            pltpu.async_copy(ibias.at[ibh_idx], ib_rows, sem),
        ]
        for j in range(FPB // ICH):
            sl = pl.ds(j * ICH, ICH)
            copies.append(
                pltpu.async_copy(ufeat_tab.at[ufidx.at[sl]], uf_rows.at[sl], sem))
            copies.append(
                pltpu.async_copy(ifeat_tab.at[ifidx.at[sl]], if_rows.at[sl], sem))
        for c in copies:
            c.wait()

        def dot_body(e, _):
            acc = uid_rows[e, pl.ds(0, L)] * iid_rows[e, pl.ds(0, L)]
            for s in range(1, D_ID // L):
                acc += uid_rows[e, pl.ds(s * L, L)] * iid_rows[e, pl.ds(s * L, L)]
            rb = e * NF
            for s in range(NF * D_F // L):
                row = rb + 2 * s + i_d8
                pu = plsc.load_gather(uf_rows, [row, i_m8])
                pv = plsc.load_gather(if_rows, [row, i_m8])
                acc += pu * pv
            tot = plsc.cumsum(acc)
            plsc.store_scatter(out_v, [zeros + (e0 + e)], tot, mask=m15)
            return 0

        lax.fori_loop(0, CB, dot_body, 0)

        def bias_body(t, _):
            r = t * L + iota
            uc = uid_idx[pl.ds(t * L, L)] & 7
            ic = iid_idx[pl.ds(t * L, L)] & 7
            bu = plsc.load_gather(ub_rows, [r, uc])
            bi = plsc.load_gather(ib_rows, [r, ic])
            sl = pl.ds(e0 + t * L, L)
            out_v[sl] = out_v[sl] + bu + bi
            return 0

        lax.fori_loop(0, CB // L, bias_body, 0, unroll=2)

    pltpu.sync_copy(out_v, out_hbm.at[pl.ds(base, CPW)])


@functools.partial(jax.jit, static_argnames=())
def kernel(users_features, items_features, user_id_table, user_feat_tables,
           user_bias, item_id_table, item_feat_tables, item_bias):
    ufeat2 = user_feat_tables.reshape(NF * V, D_F)
    ifeat2 = item_feat_tables.reshape(NF * V, D_F)
    nu8 = user_bias.shape[0] // 8
    ni8 = item_bias.shape[0] // 8
    ubias8 = user_bias[:nu8 * 8].reshape(nu8, 8)
    ibias8 = item_bias[:ni8 * 8].reshape(ni8, 8)
    mesh = plsc.VectorSubcoreMesh(core_axis_name="c", subcore_axis_name="s")
    f = pl.kernel(
        _nes_body,
        out_type=jax.ShapeDtypeStruct((B,), jnp.float32),
        mesh=mesh,
        scratch_types=[
            pltpu.VMEM((CPW, NCOL), jnp.int32),
            pltpu.VMEM((CPW, NCOL), jnp.int32),
            pltpu.VMEM((CB,), jnp.int32),
            pltpu.VMEM((CB,), jnp.int32),
            pltpu.VMEM((CB,), jnp.int32),
            pltpu.VMEM((CB,), jnp.int32),
            pltpu.VMEM((FPB,), jnp.int32),
            pltpu.VMEM((FPB,), jnp.int32),
            pltpu.VMEM((CB, D_ID), jnp.float32),
            pltpu.VMEM((CB, D_ID), jnp.float32),
            pltpu.VMEM((FPB, D_F), jnp.float32),
            pltpu.VMEM((FPB, D_F), jnp.float32),
            pltpu.VMEM((CB, 8), jnp.float32),
            pltpu.VMEM((CB, 8), jnp.float32),
            pltpu.VMEM((CPW,), jnp.float32),
            pltpu.SemaphoreType.DMA,
        ],
        compiler_params=pltpu.CompilerParams(
            needs_layout_passes=False, use_tc_tiling_on_sc=False),
    )
    return f(users_features, items_features, user_id_table, ufeat2, ubias8,
             item_id_table, ifeat2, ibias8)


# speedup vs baseline: 1.0873x; 1.0116x over previous
import functools

import jax
import jax.numpy as jnp
from jax import lax
from jax.experimental import pallas as pl
from jax.experimental.pallas import tpu as pltpu
from jax.experimental.pallas import tpu_sc as plsc

B = 16384
V = 100000
D_ID = 64
D_F = 8
NF = 12
NCOL = 1 + NF

NC = 2
NS = 16
NW = NC * NS
L = 16

CPW = B // NW
NB = 4
CB = CPW // NB
FPB = CB * NF
ICH = 128


def _nes_body(uf_hbm, if_hbm, uid_tab, ufeat_tab, ubias, iid_tab, ifeat_tab,
              ibias, out_hbm, uf_v, if_v,
              uid_idx0, iid_idx0, ubh_idx0, ibh_idx0, ufidx0, ifidx0,
              uid_rows0, iid_rows0, uf_rows0, if_rows0, ub_rows0, ib_rows0,
              uid_idx1, iid_idx1, ubh_idx1, ibh_idx1, ufidx1, ifidx1,
              uid_rows1, iid_rows1, uf_rows1, if_rows1, ub_rows1, ib_rows1,
              out_v, sem0, sem1):
    wid = lax.axis_index("s") * NC + lax.axis_index("c")
    base = wid * CPW

    iota = lax.iota(jnp.int32, L)
    zeros = iota * 0
    i_d8 = iota // 8
    i_m8 = iota % 8
    m15 = iota == 15

    bufs = [
        (uid_idx0, iid_idx0, ubh_idx0, ibh_idx0, ufidx0, ifidx0,
         uid_rows0, iid_rows0, uf_rows0, if_rows0, ub_rows0, ib_rows0, sem0),
        (uid_idx1, iid_idx1, ubh_idx1, ibh_idx1, ufidx1, ifidx1,
         uid_rows1, iid_rows1, uf_rows1, if_rows1, ub_rows1, ib_rows1, sem1),
    ]

    pltpu.sync_copy(uf_hbm.at[pl.ds(base, CPW)], uf_v)
    pltpu.sync_copy(if_hbm.at[pl.ds(base, CPW)], if_v)

    def build_and_fire(g):
        (uid_idx, iid_idx, ubh_idx, ibh_idx, ufidx, ifidx,
         uid_rows, iid_rows, uf_rows, if_rows, ub_rows, ib_rows,
         sem) = bufs[g % 2]
        e0 = g * CB

        def id_idx_body(t, _):
            r = e0 + t * L + iota
            u = plsc.load_gather(uf_v, [r, zeros])
            iv = plsc.load_gather(if_v, [r, zeros])
            uid_idx[pl.ds(t * L, L)] = u
            iid_idx[pl.ds(t * L, L)] = iv
            ubh_idx[pl.ds(t * L, L)] = u >> 3
            ibh_idx[pl.ds(t * L, L)] = iv >> 3
            return 0

        lax.fori_loop(0, CB // L, id_idx_body, 0, unroll=2)

        def f_idx_body(t, _):
            k = t * L + iota
            e = k // NF
            i = k - e * NF
            r = e0 + e
            c = 1 + i
            off = i * V
            ufidx[pl.ds(t * L, L)] = plsc.load_gather(uf_v, [r, c]) + off
            ifidx[pl.ds(t * L, L)] = plsc.load_gather(if_v, [r, c]) + off
            return 0

        lax.fori_loop(0, FPB // L, f_idx_body, 0, unroll=2)

        copies = [
            pltpu.async_copy(uid_tab.at[uid_idx], uid_rows, sem),
            pltpu.async_copy(iid_tab.at[iid_idx], iid_rows, sem),
            pltpu.async_copy(ubias.at[ubh_idx], ub_rows, sem),
            pltpu.async_copy(ibias.at[ibh_idx], ib_rows, sem),
        ]
        for j in range(FPB // ICH):
            sl = pl.ds(j * ICH, ICH)
            copies.append(
                pltpu.async_copy(ufeat_tab.at[ufidx.at[sl]], uf_rows.at[sl], sem))
            copies.append(
                pltpu.async_copy(ifeat_tab.at[ifidx.at[sl]], if_rows.at[sl], sem))
        return copies

    def compute(g):
        (uid_idx, iid_idx, ubh_idx, ibh_idx, ufidx, ifidx,
         uid_rows, iid_rows, uf_rows, if_rows, ub_rows, ib_rows,
         sem) = bufs[g % 2]
        e0 = g * CB

        def dot_body(e, _):
            acc = uid_rows[e, pl.ds(0, L)] * iid_rows[e, pl.ds(0, L)]
            for s in range(1, D_ID // L):
                acc += uid_rows[e, pl.ds(s * L, L)] * iid_rows[e, pl.ds(s * L, L)]
            rb = e * NF
            for s in range(NF * D_F // L):
                row = rb + 2 * s + i_d8
                pu = plsc.load_gather(uf_rows, [row, i_m8])
                pv = plsc.load_gather(if_rows, [row, i_m8])
                acc += pu * pv
            tot = plsc.cumsum(acc)
            plsc.store_scatter(out_v, [zeros + (e0 + e)], tot, mask=m15)
            return 0

        lax.fori_loop(0, CB, dot_body, 0)

        def bias_body(t, _):
            r = t * L + iota
            uc = uid_idx[pl.ds(t * L, L)] & 7
            ic = iid_idx[pl.ds(t * L, L)] & 7
            bu = plsc.load_gather(ub_rows, [r, uc])
            bi = plsc.load_gather(ib_rows, [r, ic])
            sl = pl.ds(e0 + t * L, L)
            out_v[sl] = out_v[sl] + bu + bi
            return 0

        lax.fori_loop(0, CB // L, bias_body, 0, unroll=2)

    copies_prev = build_and_fire(0)
    for g in range(NB):
        copies_next = build_and_fire(g + 1) if g + 1 < NB else None
        for c in copies_prev:
            c.wait()
        compute(g)
        copies_prev = copies_next

    pltpu.sync_copy(out_v, out_hbm.at[pl.ds(base, CPW)])


@functools.partial(jax.jit, static_argnames=())
def kernel(users_features, items_features, user_id_table, user_feat_tables,
           user_bias, item_id_table, item_feat_tables, item_bias):
    ufeat2 = user_feat_tables.reshape(NF * V, D_F)
    ifeat2 = item_feat_tables.reshape(NF * V, D_F)
    nu8 = user_bias.shape[0] // 8
    ni8 = item_bias.shape[0] // 8
    ubias8 = user_bias[:nu8 * 8].reshape(nu8, 8)
    ibias8 = item_bias[:ni8 * 8].reshape(ni8, 8)
    mesh = plsc.VectorSubcoreMesh(core_axis_name="c", subcore_axis_name="s")
    per_parity = [
        pltpu.VMEM((CB,), jnp.int32),
        pltpu.VMEM((CB,), jnp.int32),
        pltpu.VMEM((CB,), jnp.int32),
        pltpu.VMEM((CB,), jnp.int32),
        pltpu.VMEM((FPB,), jnp.int32),
        pltpu.VMEM((FPB,), jnp.int32),
        pltpu.VMEM((CB, D_ID), jnp.float32),
        pltpu.VMEM((CB, D_ID), jnp.float32),
        pltpu.VMEM((FPB, D_F), jnp.float32),
        pltpu.VMEM((FPB, D_F), jnp.float32),
        pltpu.VMEM((CB, 8), jnp.float32),
        pltpu.VMEM((CB, 8), jnp.float32),
    ]
    f = pl.kernel(
        _nes_body,
        out_type=jax.ShapeDtypeStruct((B,), jnp.float32),
        mesh=mesh,
        scratch_types=(
            [pltpu.VMEM((CPW, NCOL), jnp.int32),
             pltpu.VMEM((CPW, NCOL), jnp.int32)]
            + per_parity + per_parity
            + [pltpu.VMEM((CPW,), jnp.float32),
               pltpu.SemaphoreType.DMA,
               pltpu.SemaphoreType.DMA]
        ),
        compiler_params=pltpu.CompilerParams(
            needs_layout_passes=False, use_tc_tiling_on_sc=False),
    )
    return f(users_features, items_features, user_id_table, ufeat2, ubias8,
             item_id_table, ifeat2, ibias8)
